# baseline (speedup 1.0000x reference)
import functools

import jax
import jax.numpy as jnp
from jax import lax
from jax.experimental import pallas as pl
from jax.experimental.pallas import tpu as pltpu
from jax.experimental.pallas import tpu_sc as plsc

N = 10000
N_PAD = 10240
D = 128
C_OUT = 64
E = 160000
CHUNK = 128
NC = 2
NS = 16
NW = NC * NS
PADC = 1280
E_PAD = PADC * CHUNK
ROWS_PER_SUB = N_PAD // NS
CPW_AGG = PADC // NW
CPW_CNT = PADC // NS

_mesh = plsc.VectorSubcoreMesh(
    core_axis_name="c", subcore_axis_name="s", num_cores=NC, num_subcores=NS)


def _zero_acc(zf_hbm, rows_v, acc_sp, base):
  pltpu.sync_copy(zf_hbm, rows_v)
  for k in range(ROWS_PER_SUB // CHUNK):
    pltpu.sync_copy(rows_v, acc_sp.at[pl.ds(base + k * CHUNK, CHUNK)])


def _flush_acc(acc_sp, rows_v, out_hbm, c, base):
  for k in range(ROWS_PER_SUB // CHUNK):
    pltpu.sync_copy(acc_sp.at[pl.ds(base + k * CHUNK, CHUNK)], rows_v)
    pltpu.sync_copy(rows_v, out_hbm.at[c, pl.ds(base + k * CHUNK, CHUNK)])


@functools.partial(
    pl.kernel,
    out_type=jax.ShapeDtypeStruct((NC, N_PAD, D), jnp.float32),
    mesh=_mesh,
    scratch_types=[
        pltpu.VMEM((CHUNK,), jnp.int32),
        pltpu.VMEM((CHUNK,), jnp.int32),
        pltpu.VMEM((CHUNK,), jnp.int32),
        pltpu.VMEM((CHUNK,), jnp.int32),
        pltpu.VMEM((CHUNK, D), jnp.float32),
        pltpu.VMEM((CHUNK, D), jnp.float32),
        pltpu.VMEM_SHARED((N_PAD, D), jnp.float32),
        pltpu.SemaphoreType.DMA,
        pltpu.SemaphoreType.DMA,
    ],
)
def _sc_agg(x_hbm, src_hbm, dst_hbm, zf_hbm, s_out,
            sidx_a, sidx_b, didx_a, didx_b, rows_a, rows_b,
            acc_sp, gsem, ssem):
  c = lax.axis_index("c")
  sid = lax.axis_index("s")
  base = sid * ROWS_PER_SUB
  _zero_acc(zf_hbm, rows_a, acc_sp, base)
  plsc.subcore_barrier()

  wid = sid * NC + c

  def chunk_row(j):
    return j * NW + wid

  def body(j, carry):
    pltpu.sync_copy(src_hbm.at[chunk_row(j)], sidx_a)
    pltpu.sync_copy(dst_hbm.at[chunk_row(j)], didx_a)
    pltpu.async_copy(x_hbm.at[sidx_a], rows_a, gsem).wait()
    pltpu.sync_copy(rows_a, acc_sp.at[didx_a], add=True)
    return carry

  lax.fori_loop(0, CPW_AGG, body, 0)

  plsc.subcore_barrier()
  _flush_acc(acc_sp, rows_a, s_out, c, base)


CNT_INFLIGHT = 8


@functools.partial(
    pl.kernel,
    out_type=jax.ShapeDtypeStruct((NC, N_PAD, D), jnp.float32),
    mesh=_mesh,
    scratch_types=[
        pltpu.VMEM((CPW_CNT, CHUNK), jnp.int32),
        pltpu.VMEM((CHUNK, D), jnp.float32),
        pltpu.VMEM_SHARED((N_PAD, D), jnp.float32),
        pltpu.SemaphoreType.DMA,
    ],
)
def _sc_counts(dup_hbm, dtr_hbm, on_hbm, zf_hbm, cnt_out,
               didx_v, rows_v, acc_sp, ssem):
  c = lax.axis_index("c")
  sid = lax.axis_index("s")
  base = sid * ROWS_PER_SUB
  _zero_acc(zf_hbm, rows_v, acc_sp, base)
  plsc.subcore_barrier()
  pltpu.sync_copy(on_hbm, rows_v)
  cbase = sid * CPW_CNT

  def count_edges(dst_hbm):
    pltpu.sync_copy(dst_hbm.at[pl.ds(cbase, CPW_CNT)], didx_v)

    def body(j, carry):
      pltpu.async_copy(rows_v, acc_sp.at[didx_v.at[j]], ssem, add=True)

      @pl.when(j >= CNT_INFLIGHT)
      def _():
        pltpu.make_async_copy(rows_v, acc_sp.at[didx_v.at[j]], ssem).wait()
      return carry

    lax.fori_loop(0, CPW_CNT, body, 0)
    for _ in range(CNT_INFLIGHT):
      pltpu.make_async_copy(rows_v, acc_sp.at[didx_v.at[0]], ssem).wait()

  @pl.when(c == 0)
  def _():
    count_edges(dup_hbm)

  @pl.when(c == 1)
  def _():
    count_edges(dtr_hbm)

  plsc.subcore_barrier()
  _flush_acc(acc_sp, rows_v, cnt_out, c, base)


BLK_M = 2000


def _make_sage_tc(cnt_sel):
  def body(sp_ref, cnt_ref, x_ref, wl_ref, bl_ref, wr_ref, o_ref):
    s = sp_ref[0] + sp_ref[1]
    cnt = jnp.maximum(cnt_ref[0, :, 0:1], 1.0)
    mean = s / cnt
    y = jnp.dot(mean, wl_ref[...], preferred_element_type=jnp.float32)
    y = y + jnp.dot(x_ref[...], wr_ref[...],
                    preferred_element_type=jnp.float32)
    y = y + bl_ref[...]
    o_ref[...] = jnp.where(y >= 0, y, 0.01 * y)

  return pl.pallas_call(
      body,
      grid=(N // BLK_M,),
      in_specs=[
          pl.BlockSpec((NC, BLK_M, D), lambda i: (0, i, 0)),
          pl.BlockSpec((1, BLK_M, D), lambda i: (cnt_sel, i, 0)),
          pl.BlockSpec((BLK_M, D), lambda i: (i, 0)),
          pl.BlockSpec((D, D), lambda i: (0, 0)),
          pl.BlockSpec((1, D), lambda i: (0, 0)),
          pl.BlockSpec((D, D), lambda i: (0, 0)),
      ],
      out_specs=pl.BlockSpec((BLK_M, D), lambda i: (i, 0)),
      out_shape=jax.ShapeDtypeStruct((N, D), jnp.float32),
  )


_sage_tc_t = _make_sage_tc(0)
_sage_tc_u = _make_sage_tc(1)


def _final_tc_body(s2_ref, cnt_ref, x_ref, wl_ref, bl_ref, wr_ref,
                   wo_ref, bo_ref, xt_ref, o_ref):
  s = s2_ref[0] + s2_ref[1]
  cnt = jnp.maximum(cnt_ref[0, :, 0:1], 1.0)
  mean = s / cnt
  y = jnp.dot(mean, wl_ref[...], preferred_element_type=jnp.float32)
  y = y + jnp.dot(x_ref[...], wr_ref[...], preferred_element_type=jnp.float32)
  y = y + bl_ref[...]
  y = jnp.where(y >= 0, y, 0.01 * y)
  xt_ref[...] = y
  o_ref[...] = (jnp.dot(y, wo_ref[...], preferred_element_type=jnp.float32)
                + bo_ref[...])


_final_tc = pl.pallas_call(
    _final_tc_body,
    grid=(N // BLK_M,),
    in_specs=[
        pl.BlockSpec((NC, BLK_M, D), lambda i: (0, i, 0)),
        pl.BlockSpec((1, BLK_M, D), lambda i: (0, i, 0)),
        pl.BlockSpec((BLK_M, D), lambda i: (i, 0)),
        pl.BlockSpec((D, D), lambda i: (0, 0)),
        pl.BlockSpec((1, D), lambda i: (0, 0)),
        pl.BlockSpec((D, D), lambda i: (0, 0)),
        pl.BlockSpec((D, C_OUT), lambda i: (0, 0)),
        pl.BlockSpec((1, C_OUT), lambda i: (0, 0)),
    ],
    out_specs=(
        pl.BlockSpec((BLK_M, D), lambda i: (i, 0)),
        pl.BlockSpec((BLK_M, C_OUT), lambda i: (i, 0)),
    ),
    out_shape=(
        jax.ShapeDtypeStruct((N, D), jnp.float32),
        jax.ShapeDtypeStruct((N, C_OUT), jnp.float32),
    ),
)


def _pad_edges(row, is_dst):
  npad = E_PAD - E
  if is_dst:
    fill = N + (jnp.arange(npad, dtype=jnp.int32) % (N_PAD - N))
  else:
    fill = jnp.zeros((npad,), jnp.int32)
  return jnp.concatenate([row.astype(jnp.int32), fill]).reshape(PADC, CHUNK)


def kernel(x_transaction, x_user, edge_index_upt, edge_index_tru,
           Wl_0_upt, bl_0_upt, Wr_0_upt, Wl_0_tru, bl_0_tru, Wr_0_tru,
           Wl_1_upt, bl_1_upt, Wr_1_upt, Wl_1_tru, bl_1_tru, Wr_1_tru,
           W_out, b_out):
  src_up = _pad_edges(edge_index_upt[0], False)
  dst_up = _pad_edges(edge_index_upt[1], True)
  src_tr = _pad_edges(edge_index_tru[0], False)
  dst_tr = _pad_edges(edge_index_tru[1], True)
  zf = jnp.zeros((CHUNK, D), jnp.float32)
  on = jnp.ones((CHUNK, D), jnp.float32)

  cnts = _sc_counts(dst_up, dst_tr, on, zf)
  st1p = _sc_agg(x_user, src_up, dst_up, zf)
  su1p = _sc_agg(x_transaction, src_tr, dst_tr, zf)

  xt1 = _sage_tc_t(st1p, cnts, x_transaction, Wl_0_upt,
                   bl_0_upt.reshape(1, D), Wr_0_upt)
  xu1 = _sage_tc_u(su1p, cnts, x_user, Wl_0_tru,
                   bl_0_tru.reshape(1, D), Wr_0_tru)

  s2p = _sc_agg(xu1, src_up, dst_up, zf)

  xt2, out = _final_tc(s2p, cnts, xt1, Wl_1_upt, bl_1_upt.reshape(1, D),
                       Wr_1_upt, W_out, b_out.reshape(1, C_OUT))
  return (out, xt2)


# speedup vs baseline: 1.6719x; 1.6719x over previous
import functools

import jax
import jax.numpy as jnp
from jax import lax
from jax.experimental import pallas as pl
from jax.experimental.pallas import tpu as pltpu
from jax.experimental.pallas import tpu_sc as plsc

N = 10000
N_PAD = 10240
D = 128
C_OUT = 64
E = 160000
CHUNK = 128
NCHUNKS = E // CHUNK
NC = 2
NS = 16
NW = NC * NS
ROWS_PER_SUB = N_PAD // NS

_mesh = plsc.VectorSubcoreMesh(
    core_axis_name="c", subcore_axis_name="s", num_cores=NC, num_subcores=NS)


def _zero_acc(zf_hbm, rows_v, acc_sp, base):
  pltpu.sync_copy(zf_hbm, rows_v)
  for k in range(ROWS_PER_SUB // CHUNK):
    pltpu.sync_copy(rows_v, acc_sp.at[pl.ds(base + k * CHUNK, CHUNK)])


def _flush_acc(acc_sp, rows_v, out_hbm, c, base):
  for k in range(ROWS_PER_SUB // CHUNK):
    pltpu.sync_copy(acc_sp.at[pl.ds(base + k * CHUNK, CHUNK)], rows_v)
    pltpu.sync_copy(rows_v, out_hbm.at[c, pl.ds(base + k * CHUNK, CHUNK)])


@functools.partial(
    pl.kernel,
    out_type=jax.ShapeDtypeStruct((NC, N_PAD, D), jnp.float32),
    mesh=_mesh,
    scratch_types=[
        pltpu.VMEM((CHUNK,), jnp.int32),
        pltpu.VMEM((CHUNK,), jnp.int32),
        pltpu.VMEM((CHUNK, D), jnp.float32),
        pltpu.VMEM_SHARED((N_PAD, D), jnp.float32),
        pltpu.SemaphoreType.DMA,
    ],
)
def _sc_agg(x_hbm, src_hbm, dst_hbm, zf_hbm, s_out,
            sidx_v, didx_v, rows_v, acc_sp, gsem):
  c = lax.axis_index("c")
  sid = lax.axis_index("s")
  base = sid * ROWS_PER_SUB
  _zero_acc(zf_hbm, rows_v, acc_sp, base)
  plsc.subcore_barrier()

  wid = sid * NC + c

  def body(k, carry):
    chunk = k * NW + wid
    pltpu.sync_copy(src_hbm.at[chunk], sidx_v)
    pltpu.sync_copy(dst_hbm.at[chunk], didx_v)
    pltpu.async_copy(x_hbm.at[sidx_v], rows_v, gsem).wait()
    pltpu.sync_copy(rows_v, acc_sp.at[didx_v], add=True)
    return carry

  nk = (NCHUNKS - wid + NW - 1) // NW
  lax.fori_loop(0, nk, body, 0)
  plsc.subcore_barrier()
  _flush_acc(acc_sp, rows_v, s_out, c, base)


@functools.partial(
    pl.kernel,
    out_type=jax.ShapeDtypeStruct((NC, N_PAD, D), jnp.float32),
    mesh=_mesh,
    scratch_types=[
        pltpu.VMEM((CHUNK,), jnp.int32),
        pltpu.VMEM((CHUNK, D), jnp.float32),
        pltpu.VMEM_SHARED((N_PAD, D), jnp.float32),
    ],
)
def _sc_counts(dup_hbm, dtr_hbm, on_hbm, zf_hbm, cnt_out,
               didx_v, rows_v, acc_sp):
  c = lax.axis_index("c")
  sid = lax.axis_index("s")
  base = sid * ROWS_PER_SUB
  _zero_acc(zf_hbm, rows_v, acc_sp, base)
  plsc.subcore_barrier()
  pltpu.sync_copy(on_hbm, rows_v)

  def count_edges(dst_hbm):
    def body(k, carry):
      chunk = k * NS + sid
      pltpu.sync_copy(dst_hbm.at[chunk], didx_v)
      pltpu.sync_copy(rows_v, acc_sp.at[didx_v], add=True)
      return carry
    nk = (NCHUNKS - sid + NS - 1) // NS
    lax.fori_loop(0, nk, body, 0)

  @pl.when(c == 0)
  def _():
    count_edges(dup_hbm)

  @pl.when(c == 1)
  def _():
    count_edges(dtr_hbm)

  plsc.subcore_barrier()
  _flush_acc(acc_sp, rows_v, cnt_out, c, base)


BLK_M = 2000


def _make_sage_tc(cnt_sel):
  def body(sp_ref, cnt_ref, x_ref, wl_ref, bl_ref, wr_ref, o_ref):
    s = sp_ref[0] + sp_ref[1]
    cnt = jnp.maximum(cnt_ref[0, :, 0:1], 1.0)
    mean = s / cnt
    y = jnp.dot(mean, wl_ref[...], preferred_element_type=jnp.float32)
    y = y + jnp.dot(x_ref[...], wr_ref[...],
                    preferred_element_type=jnp.float32)
    y = y + bl_ref[...]
    o_ref[...] = jnp.where(y >= 0, y, 0.01 * y)

  return pl.pallas_call(
      body,
      grid=(N // BLK_M,),
      in_specs=[
          pl.BlockSpec((NC, BLK_M, D), lambda i: (0, i, 0)),
          pl.BlockSpec((1, BLK_M, D), lambda i: (cnt_sel, i, 0)),
          pl.BlockSpec((BLK_M, D), lambda i: (i, 0)),
          pl.BlockSpec((D, D), lambda i: (0, 0)),
          pl.BlockSpec((1, D), lambda i: (0, 0)),
          pl.BlockSpec((D, D), lambda i: (0, 0)),
      ],
      out_specs=pl.BlockSpec((BLK_M, D), lambda i: (i, 0)),
      out_shape=jax.ShapeDtypeStruct((N, D), jnp.float32),
  )


_sage_tc_t = _make_sage_tc(0)
_sage_tc_u = _make_sage_tc(1)


def _final_tc_body(s2_ref, cnt_ref, x_ref, wl_ref, bl_ref, wr_ref,
                   wo_ref, bo_ref, xt_ref, o_ref):
  s = s2_ref[0] + s2_ref[1]
  cnt = jnp.maximum(cnt_ref[0, :, 0:1], 1.0)
  mean = s / cnt
  y = jnp.dot(mean, wl_ref[...], preferred_element_type=jnp.float32)
  y = y + jnp.dot(x_ref[...], wr_ref[...], preferred_element_type=jnp.float32)
  y = y + bl_ref[...]
  y = jnp.where(y >= 0, y, 0.01 * y)
  xt_ref[...] = y
  o_ref[...] = (jnp.dot(y, wo_ref[...], preferred_element_type=jnp.float32)
                + bo_ref[...])


_final_tc = pl.pallas_call(
    _final_tc_body,
    grid=(N // BLK_M,),
    in_specs=[
        pl.BlockSpec((NC, BLK_M, D), lambda i: (0, i, 0)),
        pl.BlockSpec((1, BLK_M, D), lambda i: (0, i, 0)),
        pl.BlockSpec((BLK_M, D), lambda i: (i, 0)),
        pl.BlockSpec((D, D), lambda i: (0, 0)),
        pl.BlockSpec((1, D), lambda i: (0, 0)),
        pl.BlockSpec((D, D), lambda i: (0, 0)),
        pl.BlockSpec((D, C_OUT), lambda i: (0, 0)),
        pl.BlockSpec((1, C_OUT), lambda i: (0, 0)),
    ],
    out_specs=(
        pl.BlockSpec((BLK_M, D), lambda i: (i, 0)),
        pl.BlockSpec((BLK_M, C_OUT), lambda i: (i, 0)),
    ),
    out_shape=(
        jax.ShapeDtypeStruct((N, D), jnp.float32),
        jax.ShapeDtypeStruct((N, C_OUT), jnp.float32),
    ),
)


def kernel(x_transaction, x_user, edge_index_upt, edge_index_tru,
           Wl_0_upt, bl_0_upt, Wr_0_upt, Wl_0_tru, bl_0_tru, Wr_0_tru,
           Wl_1_upt, bl_1_upt, Wr_1_upt, Wl_1_tru, bl_1_tru, Wr_1_tru,
           W_out, b_out):
  src_up = edge_index_upt[0].astype(jnp.int32).reshape(NCHUNKS, CHUNK)
  dst_up = edge_index_upt[1].astype(jnp.int32).reshape(NCHUNKS, CHUNK)
  src_tr = edge_index_tru[0].astype(jnp.int32).reshape(NCHUNKS, CHUNK)
  dst_tr = edge_index_tru[1].astype(jnp.int32).reshape(NCHUNKS, CHUNK)
  zf = jnp.zeros((CHUNK, D), jnp.float32)
  on = jnp.ones((CHUNK, D), jnp.float32)

  cnts = _sc_counts(dst_up, dst_tr, on, zf)
  st1p = _sc_agg(x_user, src_up, dst_up, zf)
  su1p = _sc_agg(x_transaction, src_tr, dst_tr, zf)

  xt1 = _sage_tc_t(st1p, cnts, x_transaction, Wl_0_upt,
                   bl_0_upt.reshape(1, D), Wr_0_upt)
  xu1 = _sage_tc_u(su1p, cnts, x_user, Wl_0_tru,
                   bl_0_tru.reshape(1, D), Wr_0_tru)

  s2p = _sc_agg(xu1, src_up, dst_up, zf)

  xt2, out = _final_tc(s2p, cnts, xt1, Wl_1_upt, bl_1_upt.reshape(1, D),
                       Wr_1_upt, W_out, b_out.reshape(1, C_OUT))
  return (out, xt2)
